# baseline (speedup 1.0000x reference)
import functools

import jax
import jax.numpy as jnp
from jax import lax
from jax.experimental import pallas as pl
from jax.experimental.pallas import tpu as pltpu
from jax.experimental.pallas import tpu_sc as plsc

NC, NS, L = 2, 16, 16
NW = NC * NS


@functools.lru_cache(maxsize=None)
def _build(B, S, H):
    BW = B // NW
    CH = 64
    NCH = S // CH
    NG = CH // L
    mesh = plsc.VectorSubcoreMesh(core_axis_name="c", subcore_axis_name="s")

    @functools.partial(
        pl.kernel,
        mesh=mesh,
        compiler_params=pltpu.CompilerParams(needs_layout_passes=False),
        out_type=jax.ShapeDtypeStruct((B, S, H), jnp.float32),
        scratch_types=[
            pltpu.VMEM((BW, S), jnp.int32),
            pltpu.VMEM((BW, S), jnp.int32),
            pltpu.VMEM((H, S), jnp.float32),
            pltpu.VMEM((H // L, L), jnp.float32),
            pltpu.VMEM((H // L, L), jnp.float32),
            pltpu.VMEM((H // L, L), jnp.float32),
            pltpu.VMEM((CH, H), jnp.float32),
            pltpu.SemaphoreType.DMA,
        ],
    )
    def sc_kernel(ids_hbm, tt_hbm, word_hbm, posT_hbm, tB_hbm, gam_hbm, bet_hbm,
                  out_hbm, ids_v, tt_v, posT_v, tB_v, gam_v, bet_v, buf_v,
                  sem):
        wid = lax.axis_index("s") * NC + lax.axis_index("c")
        b_lo = wid * BW
        pltpu.sync_copy(ids_hbm.at[pl.ds(b_lo, BW)], ids_v)
        pltpu.sync_copy(tt_hbm.at[pl.ds(b_lo, BW)], tt_v)
        pltpu.sync_copy(posT_hbm, posT_v)
        pltpu.sync_copy(tB_hbm, tB_v)
        pltpu.sync_copy(gam_hbm, gam_v)
        pltpu.sync_copy(bet_hbm, bet_v)
        rid = lax.iota(jnp.int32, L)
        zero = jnp.zeros((L,), jnp.float32)

        def chunk_body(t, carry):
            bl = t // NCH
            s0 = (t % NCH) * CH
            pltpu.async_copy(
                word_hbm.at[ids_v.at[bl, pl.ds(s0, CH)]], buf_v, sem).wait()
            ttf = [tt_v[bl, pl.ds(s0 + g * L, L)].astype(jnp.float32)
                   for g in range(NG)]

            def p1(h, c):
                sms, sqs = c
                hs = jnp.full((L,), h, jnp.int32)
                tb = plsc.load_gather(tB_v, [hs >> 4, hs & 15])
                n_sms, n_sqs = [], []
                for g in range(NG):
                    w = plsc.load_gather(buf_v, [rid + g * L, hs])
                    p = posT_v[h, pl.ds(s0 + g * L, L)]
                    cv = w + p + ttf[g] * tb
                    plsc.store_scatter(buf_v, [rid + g * L, hs], cv)
                    n_sms.append(sms[g] + cv)
                    n_sqs.append(sqs[g] + cv * cv)
                return (tuple(n_sms), tuple(n_sqs))

            sms, sqs = lax.fori_loop(
                0, H, p1, (tuple([zero] * NG), tuple([zero] * NG)))

            means, rstds = [], []
            for g in range(NG):
                mean = sms[g] * (1.0 / H)
                var = sqs[g] * (1.0 / H) - mean * mean
                x = var + 1e-12
                i = plsc.bitcast(x, jnp.int32)
                y = plsc.bitcast(jnp.int32(0x5F3759DF) - (i >> 1), jnp.float32)
                for _ in range(3):
                    y = y * (1.5 - 0.5 * x * y * y)
                means.append(mean)
                rstds.append(y)

            def p2(h, c):
                hs = jnp.full((L,), h, jnp.int32)
                ga = plsc.load_gather(gam_v, [hs >> 4, hs & 15])
                be = plsc.load_gather(bet_v, [hs >> 4, hs & 15])
                for g in range(NG):
                    cv = plsc.load_gather(buf_v, [rid + g * L, hs])
                    a = rstds[g] * ga
                    b = be - means[g] * a
                    plsc.store_scatter(buf_v, [rid + g * L, hs], cv * a + b)
                return c

            lax.fori_loop(0, H, p2, 0)
            pltpu.sync_copy(buf_v, out_hbm.at[b_lo + bl, pl.ds(s0, CH)])
            return carry

        lax.fori_loop(0, BW * NCH, chunk_body, 0)

    return sc_kernel


def kernel(input_ids, token_type_ids, word_emb, pos_emb, type_emb, gamma, beta):
    B, S = input_ids.shape
    H = word_emb.shape[1]
    ids = input_ids.astype(jnp.int32)
    tt = token_type_ids.astype(jnp.int32)
    posT = pos_emb[:S].astype(jnp.float32).T + type_emb[0][:, None]
    tB = (type_emb[1] - type_emb[0]).reshape(H // 16, 16)
    fn = _build(B, S, H)
    return fn(ids, tt, word_emb.astype(jnp.float32), posT, tB,
              gamma.astype(jnp.float32).reshape(H // 16, 16),
              beta.astype(jnp.float32).reshape(H // 16, 16))
